# baseline (speedup 1.0000x reference)
import functools

import jax
import jax.numpy as jnp
from jax import lax
from jax.experimental import pallas as pl
from jax.experimental.pallas import tpu as pltpu
from jax.experimental.pallas import tpu_sc as plsc

B, N, D, H, R = 4096, 200, 16, 16, 8
B0 = 3072
BBLK = 64
TB = BBLK * N
GRID0 = B0 // BBLK

SC_B = B - B0
NPAIR = SC_B // 2
NW = 32
PAIRS_PER_W = NPAIR // NW
P0 = B0 // 2
CH = 25


def _leaky(x):
    return jnp.maximum(x, 0.01 * x)


def _tc_kernel(u_ref, i_ref, a_ref, o_ref, s_ref,
               wui1a_ref, wui1b_ref, bui1_ref, wui2_ref, bui2_ref,
               w1a_ref, w1b_ref, b1_ref, w2_ref, b2_ref, gt_ref,
               out_ref):
    f32 = jnp.float32
    h_ui = _leaky(jnp.dot(u_ref[...], wui1a_ref[...], preferred_element_type=f32)
                  + jnp.dot(i_ref[...], wui1b_ref[...], preferred_element_type=f32)
                  + bui1_ref[...])
    ui_emb = _leaky(jnp.dot(h_ui, wui2_ref[...], preferred_element_type=f32)
                    + bui2_ref[...])
    ui_t = jnp.concatenate([ui_emb] * R, axis=-1)
    h_all = _leaky(jnp.dot(a_ref[...], w1a_ref[...], preferred_element_type=f32)
                   + jnp.dot(o_ref[...], w1b_ref[...], preferred_element_type=f32)
                   + b1_ref[...])
    out_all = _leaky(jnp.dot(h_all, w2_ref[...], preferred_element_type=f32)
                     + b2_ref[...])
    ui_b = jnp.broadcast_to(ui_t[:, None, :], (BBLK, N, R * H)).reshape(TB, R * H)
    ou = out_all * ui_b
    scores_t = jax.lax.dot_general(
        gt_ref[...], ou, (((1,), (1,)), ((), ())), preferred_element_type=f32)
    s_row = s_ref[0]
    oh = jax.lax.broadcasted_iota(jnp.int32, (R, TB), 0) == s_row
    out_ref[...] = jnp.sum(jnp.where(oh, scores_t, 0.0), axis=0,
                           keepdims=True)[None]


@jax.jit
def _run_tc(u_emb, i_emb, a2, o2, s3,
            wui1a, wui1b, bui1, Wui2, bui2, w1a, w1b, b1_all, w2_bd, b2_all, gt):
    full = lambda shape: pl.BlockSpec(shape, lambda b: (0,) * len(shape))
    out3 = pl.pallas_call(
        _tc_kernel,
        grid=(GRID0,),
        in_specs=[
            pl.BlockSpec((BBLK, D), lambda b: (b, 0)),
            pl.BlockSpec((BBLK, D), lambda b: (b, 0)),
            pl.BlockSpec((TB, D), lambda b: (b, 0)),
            pl.BlockSpec((TB, D), lambda b: (b, 0)),
            pl.BlockSpec((1, 1, TB), lambda b: (b, 0, 0)),
            full((D, H)), full((D, H)), full((H,)), full((H, H)), full((H,)),
            full((D, R * H)), full((D, R * H)), full((R * H,)),
            full((R * H, R * H)), full((R * H,)), full((R, R * H)),
        ],
        out_specs=pl.BlockSpec((1, 1, TB), lambda b: (b, 0, 0)),
        out_shape=jax.ShapeDtypeStruct((GRID0, 1, TB), jnp.float32),
        compiler_params=pltpu.CompilerParams(
            dimension_semantics=("arbitrary",),
        ),
    )(u_emb, i_emb, a2, o2, s3,
      wui1a, wui1b, bui1, Wui2, bui2, w1a, w1b, b1_all, w2_bd, b2_all, gt)
    return out3.reshape(B0, N)


def _sc_body(a3, o3, s2, uin2, w1af, w1bf, w2f, b1f, b2f,
             wui1f, wui2f, bui1f, bui2f, out_hbm,
             wv1a, wv1b, wv2, bv1, bv2, wu1, wu2, bu1, bu2,
             av, ov, sv, uv, hscr, predv_ref):
    i32 = jnp.int32
    lane = lax.broadcasted_iota(i32, (16,), 0)
    wid = lax.axis_index("s") * 2 + lax.axis_index("c")

    pltpu.sync_copy(w1af, wv1a)
    pltpu.sync_copy(w1bf, wv1b)
    pltpu.sync_copy(w2f, wv2)
    pltpu.sync_copy(b1f, bv1)
    pltpu.sync_copy(b2f, bv2)
    pltpu.sync_copy(wui1f, wu1)
    pltpu.sync_copy(wui2f, wu2)
    pltpu.sync_copy(bui1f, bu1)
    pltpu.sync_copy(bui2f, bu2)

    def ui_mlp(off):
        x0 = uv[pl.ds(off, H)]
        x1 = uv[pl.ds(off + H, H)]
        acc = bu1[:]
        for k in range(D):
            acc = acc + x0[k] * wu1[pl.ds(k * H, H)]
        for k in range(D):
            acc = acc + x1[k] * wu1[pl.ds((D + k) * H, H)]
        h1 = _leaky(acc)
        acc2 = bu2[:]
        for k in range(H):
            acc2 = acc2 + h1[k] * wu2[pl.ds(k * H, H)]
        return _leaky(acc2)

    def pair_body(p_local, carry):
        p = P0 + wid * PAIRS_PER_W + p_local
        pltpu.sync_copy(a3.at[p], av)
        pltpu.sync_copy(o3.at[p], ov)
        pltpu.sync_copy(s2.at[p], sv)
        pltpu.sync_copy(uin2.at[p], uv)
        ui0 = ui_mlp(0)
        ui1 = ui_mlp(2 * D)

        def chunk_body(j, carry2):
            base = j * 16
            s_chunk = sv[pl.ds(base, 16)]
            predv = jnp.zeros((16,), jnp.float32)
            for jj in range(16):
                n = base + jj
                r = s_chunk[jj]
                a_row = av[pl.ds(n * D, D)]
                o_row = ov[pl.ds(n * D, D)]
                acc = bv1[pl.ds(r * H, H)]
                w1base = r * (D * H)
                for k in range(D):
                    acc = acc + a_row[k] * wv1a[pl.ds(w1base + k * H, H)]
                for k in range(D):
                    acc = acc + o_row[k] * wv1b[pl.ds(w1base + k * H, H)]
                h1 = _leaky(acc)
                acc2 = bv2[pl.ds(r * H, H)]
                w2base = r * (H * H)
                for k in range(H):
                    acc2 = acc2 + h1[k] * wv2[pl.ds(w2base + k * H, H)]
                out_t = _leaky(acc2)
                c0 = (n < N).astype(jnp.float32)
                ui_sel = ui0 * c0 + ui1 * (1.0 - c0)
                prod = out_t * ui_sel
                pt = prod[0]
                for k in range(1, H):
                    pt = pt + prod[k]
                dd = (lane - jj) * (lane - jj)
                ohv = (1 - jnp.minimum(dd, 1)).astype(jnp.float32)
                predv = predv + pt * ohv
            predv_ref[pl.ds(base, 16)] = predv
            return carry2

        lax.fori_loop(0, CH, chunk_body, 0)
        pltpu.sync_copy(predv_ref, out_hbm.at[p - P0])
        return carry

    lax.fori_loop(0, PAIRS_PER_W, pair_body, 0)


@jax.jit
def _run_sc(a3, o3, s2, uin2, w1af, w1bf, w2f, b1f, b2f,
            wui1f, wui2f, bui1f, bui2f):
    mesh = plsc.VectorSubcoreMesh(core_axis_name="c", subcore_axis_name="s")
    f32 = jnp.float32
    kern = functools.partial(
        pl.kernel, mesh=mesh,
        out_type=jax.ShapeDtypeStruct((NPAIR, 2 * N), f32),
        scratch_types=[
            pltpu.VMEM((R * D * H,), f32),
            pltpu.VMEM((R * D * H,), f32),
            pltpu.VMEM((R * H * H,), f32),
            pltpu.VMEM((R * H,), f32),
            pltpu.VMEM((R * H,), f32),
            pltpu.VMEM((2 * D * H,), f32),
            pltpu.VMEM((H * H,), f32),
            pltpu.VMEM((H,), f32),
            pltpu.VMEM((H,), f32),
            pltpu.VMEM((2 * N * D,), f32),
            pltpu.VMEM((2 * N * D,), f32),
            pltpu.VMEM((2 * N,), jnp.int32),
            pltpu.VMEM((4 * D,), f32),
            pltpu.VMEM((H,), f32),
            pltpu.VMEM((2 * N,), f32),
        ],
    )(_sc_body)
    return kern(a3, o3, s2, uin2, w1af, w1bf, w2f, b1f, b2f,
                wui1f, wui2f, bui1f, bui2f)


@jax.jit
def kernel(u_emb, i_emb, a_emb, o_emb, s,
           Wui1, bui1, Wui2, bui2, Wao1, bao1, Wao2, bao2):
    f32 = jnp.float32
    w1_all = jnp.transpose(Wao1, (1, 0, 2)).reshape(2 * D, R * H)
    b1_all = bao1.reshape(R * H)
    eye = jnp.eye(R, dtype=Wao2.dtype)
    w2_bd = jnp.einsum('rkj,rq->rkqj', Wao2, eye).reshape(R * H, R * H)
    b2_all = bao2.reshape(R * H)
    gt = jnp.repeat(jnp.eye(R, dtype=f32), H, axis=0).T
    a2 = a_emb.reshape(B * N, D)
    o2 = o_emb.reshape(B * N, D)
    s3 = s.reshape(B // BBLK, 1, TB)
    tc_out = _run_tc(u_emb, i_emb, a2, o2, s3[:GRID0],
                     Wui1[:D], Wui1[D:], bui1, Wui2, bui2,
                     w1_all[:D], w1_all[D:], b1_all, w2_bd, b2_all, gt)

    a3 = a_emb.reshape(B // 2, 2 * N * D)
    o3 = o_emb.reshape(B // 2, 2 * N * D)
    s2 = s.reshape(B // 2, 2 * N)
    uin2 = jnp.concatenate([u_emb, i_emb], axis=-1).reshape(B // 2, 4 * D)
    w1af = Wao1[:, :D, :].reshape(-1)
    w1bf = Wao1[:, D:, :].reshape(-1)
    w2f = Wao2.reshape(-1)
    b1f = bao1.reshape(-1)
    b2f = bao2.reshape(-1)
    wui1af = Wui1.reshape(-1)
    wui2f = Wui2.reshape(-1)
    sc_out = _run_sc(a3, o3, s2, uin2, w1af, w1bf, w2f, b1f, b2f,
                     wui1af, wui2f, bui1, bui2)
    return jnp.concatenate([tc_out, sc_out.reshape(SC_B, N)], axis=0)


# speedup vs baseline: 2.2134x; 2.2134x over previous
import functools

import jax
import jax.numpy as jnp
from jax import lax
from jax.experimental import pallas as pl
from jax.experimental.pallas import tpu as pltpu
from jax.experimental.pallas import tpu_sc as plsc

B, N, D, H, R = 4096, 200, 16, 16, 8
B0 = 3968
BBLK = 64
TB = BBLK * N
GRID0 = B0 // BBLK

SC_B = B - B0
NPAIR = SC_B // 2
NW = 32
PAIRS_PER_W = NPAIR // NW
P0 = B0 // 2
CH = 25


def _leaky(x):
    return jnp.maximum(x, 0.01 * x)


def _tc_kernel(u_ref, i_ref, a_ref, o_ref, s_ref,
               wui1a_ref, wui1b_ref, bui1_ref, wui2_ref, bui2_ref,
               w1a_ref, w1b_ref, b1_ref, w2_ref, b2_ref, gt_ref,
               out_ref):
    f32 = jnp.float32
    h_ui = _leaky(jnp.dot(u_ref[...], wui1a_ref[...], preferred_element_type=f32)
                  + jnp.dot(i_ref[...], wui1b_ref[...], preferred_element_type=f32)
                  + bui1_ref[...])
    ui_emb = _leaky(jnp.dot(h_ui, wui2_ref[...], preferred_element_type=f32)
                    + bui2_ref[...])
    ui_t = jnp.concatenate([ui_emb] * R, axis=-1)
    h_all = _leaky(jnp.dot(a_ref[...], w1a_ref[...], preferred_element_type=f32)
                   + jnp.dot(o_ref[...], w1b_ref[...], preferred_element_type=f32)
                   + b1_ref[...])
    out_all = _leaky(jnp.dot(h_all, w2_ref[...], preferred_element_type=f32)
                     + b2_ref[...])
    ui_b = jnp.broadcast_to(ui_t[:, None, :], (BBLK, N, R * H)).reshape(TB, R * H)
    ou = out_all * ui_b
    scores_t = jax.lax.dot_general(
        gt_ref[...], ou, (((1,), (1,)), ((), ())), preferred_element_type=f32)
    s_row = s_ref[0]
    oh = jax.lax.broadcasted_iota(jnp.int32, (R, TB), 0) == s_row
    out_ref[...] = jnp.sum(jnp.where(oh, scores_t, 0.0), axis=0,
                           keepdims=True)[None]


@jax.jit
def _run_tc(u_emb, i_emb, a2, o2, s3,
            wui1a, wui1b, bui1, Wui2, bui2, w1a, w1b, b1_all, w2_bd, b2_all, gt):
    full = lambda shape: pl.BlockSpec(shape, lambda b: (0,) * len(shape))
    out3 = pl.pallas_call(
        _tc_kernel,
        grid=(GRID0,),
        in_specs=[
            pl.BlockSpec((BBLK, D), lambda b: (b, 0)),
            pl.BlockSpec((BBLK, D), lambda b: (b, 0)),
            pl.BlockSpec((TB, D), lambda b: (b, 0)),
            pl.BlockSpec((TB, D), lambda b: (b, 0)),
            pl.BlockSpec((1, 1, TB), lambda b: (b, 0, 0)),
            full((D, H)), full((D, H)), full((H,)), full((H, H)), full((H,)),
            full((D, R * H)), full((D, R * H)), full((R * H,)),
            full((R * H, R * H)), full((R * H,)), full((R, R * H)),
        ],
        out_specs=pl.BlockSpec((1, 1, TB), lambda b: (b, 0, 0)),
        out_shape=jax.ShapeDtypeStruct((GRID0, 1, TB), jnp.float32),
        compiler_params=pltpu.CompilerParams(
            dimension_semantics=("arbitrary",),
        ),
    )(u_emb, i_emb, a2, o2, s3,
      wui1a, wui1b, bui1, Wui2, bui2, w1a, w1b, b1_all, w2_bd, b2_all, gt)
    return out3.reshape(B0, N)


def _sc_body(a3, o3, s2, uin2, w1af, w1bf, w2f, b1f, b2f,
             wui1f, wui2f, bui1f, bui2f, out_hbm,
             wv1a, wv1b, wv2, bv1, bv2, wu1, wu2, bu1, bu2,
             av, ov, sv, uv, hscr, predv_ref):
    i32 = jnp.int32
    lane = lax.broadcasted_iota(i32, (16,), 0)
    wid = lax.axis_index("s") * 2 + lax.axis_index("c")

    pltpu.sync_copy(w1af, wv1a)
    pltpu.sync_copy(w1bf, wv1b)
    pltpu.sync_copy(w2f, wv2)
    pltpu.sync_copy(b1f, bv1)
    pltpu.sync_copy(b2f, bv2)
    pltpu.sync_copy(wui1f, wu1)
    pltpu.sync_copy(wui2f, wu2)
    pltpu.sync_copy(bui1f, bu1)
    pltpu.sync_copy(bui2f, bu2)

    def ui_mlp(off):
        x0 = uv[pl.ds(off, H)]
        x1 = uv[pl.ds(off + H, H)]
        acc = bu1[:]
        for k in range(D):
            acc = acc + x0[k] * wu1[pl.ds(k * H, H)]
        for k in range(D):
            acc = acc + x1[k] * wu1[pl.ds((D + k) * H, H)]
        h1 = _leaky(acc)
        acc2 = bu2[:]
        for k in range(H):
            acc2 = acc2 + h1[k] * wu2[pl.ds(k * H, H)]
        return _leaky(acc2)

    def pair_body(p_local, carry):
        p = P0 + wid * PAIRS_PER_W + p_local
        pltpu.sync_copy(a3.at[p], av)
        pltpu.sync_copy(o3.at[p], ov)
        pltpu.sync_copy(s2.at[p], sv)
        pltpu.sync_copy(uin2.at[p], uv)
        ui0 = ui_mlp(0)
        ui1 = ui_mlp(2 * D)

        def chunk_body(j, carry2):
            base = j * 16
            s_chunk = sv[pl.ds(base, 16)]
            predv = jnp.zeros((16,), jnp.float32)
            for jj in range(16):
                n = base + jj
                r = s_chunk[jj]
                a_row = av[pl.ds(n * D, D)]
                o_row = ov[pl.ds(n * D, D)]
                acc = bv1[pl.ds(r * H, H)]
                w1base = r * (D * H)
                for k in range(D):
                    acc = acc + a_row[k] * wv1a[pl.ds(w1base + k * H, H)]
                for k in range(D):
                    acc = acc + o_row[k] * wv1b[pl.ds(w1base + k * H, H)]
                h1 = _leaky(acc)
                acc2 = bv2[pl.ds(r * H, H)]
                w2base = r * (H * H)
                for k in range(H):
                    acc2 = acc2 + h1[k] * wv2[pl.ds(w2base + k * H, H)]
                out_t = _leaky(acc2)
                c0 = (n < N).astype(jnp.float32)
                ui_sel = ui0 * c0 + ui1 * (1.0 - c0)
                prod = out_t * ui_sel
                pt = prod[0]
                for k in range(1, H):
                    pt = pt + prod[k]
                dd = (lane - jj) * (lane - jj)
                ohv = (1 - jnp.minimum(dd, 1)).astype(jnp.float32)
                predv = predv + pt * ohv
            predv_ref[pl.ds(base, 16)] = predv
            return carry2

        lax.fori_loop(0, CH, chunk_body, 0)
        pltpu.sync_copy(predv_ref, out_hbm.at[p - P0])
        return carry

    lax.fori_loop(0, PAIRS_PER_W, pair_body, 0)


@jax.jit
def _run_sc(a3, o3, s2, uin2, w1af, w1bf, w2f, b1f, b2f,
            wui1f, wui2f, bui1f, bui2f):
    mesh = plsc.VectorSubcoreMesh(core_axis_name="c", subcore_axis_name="s")
    f32 = jnp.float32
    kern = functools.partial(
        pl.kernel, mesh=mesh,
        out_type=jax.ShapeDtypeStruct((NPAIR, 2 * N), f32),
        scratch_types=[
            pltpu.VMEM((R * D * H,), f32),
            pltpu.VMEM((R * D * H,), f32),
            pltpu.VMEM((R * H * H,), f32),
            pltpu.VMEM((R * H,), f32),
            pltpu.VMEM((R * H,), f32),
            pltpu.VMEM((2 * D * H,), f32),
            pltpu.VMEM((H * H,), f32),
            pltpu.VMEM((H,), f32),
            pltpu.VMEM((H,), f32),
            pltpu.VMEM((2 * N * D,), f32),
            pltpu.VMEM((2 * N * D,), f32),
            pltpu.VMEM((2 * N,), jnp.int32),
            pltpu.VMEM((4 * D,), f32),
            pltpu.VMEM((H,), f32),
            pltpu.VMEM((2 * N,), f32),
        ],
    )(_sc_body)
    return kern(a3, o3, s2, uin2, w1af, w1bf, w2f, b1f, b2f,
                wui1f, wui2f, bui1f, bui2f)


@jax.jit
def kernel(u_emb, i_emb, a_emb, o_emb, s,
           Wui1, bui1, Wui2, bui2, Wao1, bao1, Wao2, bao2):
    f32 = jnp.float32
    w1_all = jnp.transpose(Wao1, (1, 0, 2)).reshape(2 * D, R * H)
    b1_all = bao1.reshape(R * H)
    eye = jnp.eye(R, dtype=Wao2.dtype)
    w2_bd = jnp.einsum('rkj,rq->rkqj', Wao2, eye).reshape(R * H, R * H)
    b2_all = bao2.reshape(R * H)
    gt = jnp.repeat(jnp.eye(R, dtype=f32), H, axis=0).T
    a2 = a_emb.reshape(B * N, D)
    o2 = o_emb.reshape(B * N, D)
    s3 = s.reshape(B // BBLK, 1, TB)
    tc_out = _run_tc(u_emb, i_emb, a2, o2, s3[:GRID0],
                     Wui1[:D], Wui1[D:], bui1, Wui2, bui2,
                     w1_all[:D], w1_all[D:], b1_all, w2_bd, b2_all, gt)

    a3 = a_emb.reshape(B // 2, 2 * N * D)
    o3 = o_emb.reshape(B // 2, 2 * N * D)
    s2 = s.reshape(B // 2, 2 * N)
    uin2 = jnp.concatenate([u_emb, i_emb], axis=-1).reshape(B // 2, 4 * D)
    w1af = Wao1[:, :D, :].reshape(-1)
    w1bf = Wao1[:, D:, :].reshape(-1)
    w2f = Wao2.reshape(-1)
    b1f = bao1.reshape(-1)
    b2f = bao2.reshape(-1)
    wui1af = Wui1.reshape(-1)
    wui2f = Wui2.reshape(-1)
    sc_out = _run_sc(a3, o3, s2, uin2, w1af, w1bf, w2f, b1f, b2f,
                     wui1af, wui2f, bui1, bui2)
    return jnp.concatenate([tc_out, sc_out.reshape(SC_B, N)], axis=0)


# speedup vs baseline: 4.2473x; 1.9189x over previous
import jax
import jax.numpy as jnp
from jax.experimental import pallas as pl
from jax.experimental.pallas import tpu as pltpu

B, N, D, H, R = 4096, 200, 16, 16, 8
BBLK = 64
TB = BBLK * N
GRID = B // BBLK


def _leaky(x):
    return jnp.maximum(x, 0.01 * x)


def _block_kernel(u_ref, i_ref, a_ref, o_ref, s_ref,
                  wui1a_ref, wui1b_ref, bui1_ref, wui2_ref, bui2_ref,
                  w1a_ref, w1b_ref, b1_ref, w2_ref, b2_ref, gt_ref,
                  out_ref):
    f32 = jnp.float32
    h_ui = _leaky(jnp.dot(u_ref[...], wui1a_ref[...], preferred_element_type=f32)
                  + jnp.dot(i_ref[...], wui1b_ref[...], preferred_element_type=f32)
                  + bui1_ref[...])
    ui_emb = _leaky(jnp.dot(h_ui, wui2_ref[...], preferred_element_type=f32)
                    + bui2_ref[...])
    ui_t = jnp.concatenate([ui_emb] * R, axis=-1)

    h_all = _leaky(jnp.dot(a_ref[...], w1a_ref[...], preferred_element_type=f32)
                   + jnp.dot(o_ref[...], w1b_ref[...], preferred_element_type=f32)
                   + b1_ref[...])
    out_all = _leaky(jnp.dot(h_all, w2_ref[...], preferred_element_type=f32)
                     + b2_ref[...])

    ui_b = jnp.broadcast_to(ui_t[:, None, :], (BBLK, N, R * H)).reshape(TB, R * H)
    ou = out_all * ui_b

    scores_t = jax.lax.dot_general(
        gt_ref[...], ou, (((1,), (1,)), ((), ())),
        preferred_element_type=f32)

    s_row = s_ref[0]
    oh = jax.lax.broadcasted_iota(jnp.int32, (R, TB), 0) == s_row
    out_ref[...] = jnp.sum(jnp.where(oh, scores_t, 0.0), axis=0,
                           keepdims=True)[None]


@jax.jit
def _run(u_emb, i_emb, a2, o2, s3,
         wui1a, wui1b, bui1, Wui2, bui2, w1a, w1b, b1_all, w2_bd, b2_all, gt):
    full = lambda shape: pl.BlockSpec(shape, lambda b: (0,) * len(shape))
    out3 = pl.pallas_call(
        _block_kernel,
        grid=(GRID,),
        in_specs=[
            pl.BlockSpec((BBLK, D), lambda b: (b, 0)),
            pl.BlockSpec((BBLK, D), lambda b: (b, 0)),
            pl.BlockSpec((TB, D), lambda b: (b, 0)),
            pl.BlockSpec((TB, D), lambda b: (b, 0)),
            pl.BlockSpec((1, 1, TB), lambda b: (b, 0, 0)),
            full((D, H)), full((D, H)), full((H,)), full((H, H)), full((H,)),
            full((D, R * H)), full((D, R * H)), full((R * H,)),
            full((R * H, R * H)), full((R * H,)), full((R, R * H)),
        ],
        out_specs=pl.BlockSpec((1, 1, TB), lambda b: (b, 0, 0)),
        out_shape=jax.ShapeDtypeStruct((GRID, 1, TB), jnp.float32),
        compiler_params=pltpu.CompilerParams(
            dimension_semantics=("arbitrary",),
        ),
    )(u_emb, i_emb, a2, o2, s3,
      wui1a, wui1b, bui1, Wui2, bui2, w1a, w1b, b1_all, w2_bd, b2_all, gt)
    return out3.reshape(B, N)


def kernel(u_emb, i_emb, a_emb, o_emb, s,
           Wui1, bui1, Wui2, bui2, Wao1, bao1, Wao2, bao2):
    w1_all = jnp.transpose(Wao1, (1, 0, 2)).reshape(2 * D, R * H)
    b1_all = bao1.reshape(R * H)
    eye = jnp.eye(R, dtype=Wao2.dtype)
    w2_bd = jnp.einsum('rkj,rq->rkqj', Wao2, eye).reshape(R * H, R * H)
    b2_all = bao2.reshape(R * H)
    gt = jnp.repeat(jnp.eye(R, dtype=jnp.float32), H, axis=0).T
    a2 = a_emb.reshape(B * N, D)
    o2 = o_emb.reshape(B * N, D)
    s3 = s.reshape(GRID, 1, TB)
    return _run(u_emb, i_emb, a2, o2, s3,
                Wui1[:D], Wui1[D:], bui1, Wui2, bui2,
                w1_all[:D], w1_all[D:], b1_all, w2_bd, b2_all, gt)
